# baseline (speedup 1.0000x reference)
import functools
import math

import jax
import jax.numpy as jnp
from jax import lax
from jax.experimental import pallas as pl
from jax.experimental.pallas import tpu as pltpu
from jax.experimental.pallas import tpu_sc as plsc

D_MODEL = 64
SCALE = math.sqrt(D_MODEL)
NUM_CORES = 2
NUM_SUBCORES = 16
NUM_WORKERS = NUM_CORES * NUM_SUBCORES
LANES = 16


def _embed_sc(idx, table, chunk, n_chunks):
    b_per_w = chunk * n_chunks
    total = b_per_w * NUM_WORKERS
    mesh = plsc.VectorSubcoreMesh(core_axis_name="c", subcore_axis_name="s")

    @functools.partial(
        pl.kernel,
        mesh=mesh,
        out_type=jax.ShapeDtypeStruct((total, D_MODEL), jnp.float32),
        scratch_types=[
            pltpu.VMEM((chunk,), jnp.int32),
            pltpu.VMEM((chunk, D_MODEL), jnp.float32),
            pltpu.SemaphoreType.DMA,
        ],
        compiler_params=pltpu.CompilerParams(use_tc_tiling_on_sc=False),
    )
    def k(idx_hbm, table_hbm, out_hbm, idx_v, rows_v, sem):
        wid = lax.axis_index("s") * NUM_CORES + lax.axis_index("c")
        base = wid * b_per_w

        def chunk_body(ci, carry):
            off = base + ci * chunk
            pltpu.sync_copy(idx_hbm.at[pl.ds(off, chunk)], idx_v)
            pltpu.async_copy(table_hbm.at[idx_v], rows_v, sem).wait()

            def row_body(r, c):
                for j in range(D_MODEL // LANES):
                    sl = pl.ds(j * LANES, LANES)
                    rows_v[r, sl] = rows_v[r, sl] * SCALE
                return c

            lax.fori_loop(0, chunk, row_body, 0)
            pltpu.sync_copy(rows_v, out_hbm.at[pl.ds(off, chunk)])
            return carry

        lax.fori_loop(0, n_chunks, chunk_body, 0)

    return k(idx, table)


def kernel(x, table):
    rows, cols = x.shape
    total = rows * cols
    idx = x.reshape(total).astype(jnp.int32)
    chunk = 800
    n_chunks = total // (NUM_WORKERS * chunk)
    out = _embed_sc(idx, table, chunk, n_chunks)
    return out.reshape(rows, cols, D_MODEL)


# speedup vs baseline: 1.1163x; 1.1163x over previous
import functools
import math

import jax
import jax.numpy as jnp
from jax import lax
from jax.experimental import pallas as pl
from jax.experimental.pallas import tpu as pltpu
from jax.experimental.pallas import tpu_sc as plsc

D_MODEL = 64
SCALE = math.sqrt(D_MODEL)
NUM_CORES = 2
NUM_SUBCORES = 16
NUM_WORKERS = NUM_CORES * NUM_SUBCORES
LANES = 16
VPR = D_MODEL // LANES
ROW_UNROLL = 4


def _embed_sc(idx, table, chunk, n_chunks):
    b_per_w = chunk * n_chunks
    total = b_per_w * NUM_WORKERS
    mesh = plsc.VectorSubcoreMesh(core_axis_name="c", subcore_axis_name="s")

    @functools.partial(
        pl.kernel,
        mesh=mesh,
        out_type=jax.ShapeDtypeStruct((total, D_MODEL), jnp.float32),
        scratch_types=[
            pltpu.VMEM((b_per_w,), jnp.int32),
            pltpu.VMEM((chunk, D_MODEL), jnp.float32),
            pltpu.VMEM((chunk, D_MODEL), jnp.float32),
            pltpu.VMEM((chunk, D_MODEL), jnp.float32),
            pltpu.VMEM((chunk, D_MODEL), jnp.float32),
            pltpu.SemaphoreType.DMA,
            pltpu.SemaphoreType.DMA,
            pltpu.SemaphoreType.DMA,
            pltpu.SemaphoreType.DMA,
        ],
        compiler_params=pltpu.CompilerParams(use_tc_tiling_on_sc=False),
    )
    def k(idx_hbm, table_hbm, out_hbm, idx_v, in0, in1, out0, out1,
          g0, g1, w0, w1):
        wid = lax.axis_index("s") * NUM_CORES + lax.axis_index("c")
        base = wid * b_per_w

        ins = (in0, in1)
        outs = (out0, out1)
        gsems = (g0, g1)
        wsems = (w0, w1)

        pltpu.sync_copy(idx_hbm.at[pl.ds(base, b_per_w)], idx_v)

        def start_gather(ci, b):
            pltpu.async_copy(
                table_hbm.at[idx_v.at[pl.ds(ci * chunk, chunk)]],
                ins[b], gsems[b])

        def scale(b):
            src = ins[b]
            dst = outs[b]

            def body(r, c):
                rr = r * ROW_UNROLL
                for u in range(ROW_UNROLL):
                    for j in range(VPR):
                        sl = pl.ds(j * LANES, LANES)
                        dst[rr + u, sl] = src[rr + u, sl] * SCALE
                return c

            lax.fori_loop(0, chunk // ROW_UNROLL, body, 0)

        start_gather(0, 0)
        start_gather(1, 1)

        def chunk_body(ci2, carry):
            for b in range(2):
                ci = ci2 * 2 + b
                pltpu.make_async_copy(
                    table_hbm.at[idx_v.at[pl.ds(0, chunk)]],
                    ins[b], gsems[b]).wait()
                @pl.when(ci >= 2)
                def _():
                    pltpu.make_async_copy(
                        outs[b], out_hbm.at[pl.ds(base, chunk)],
                        wsems[b]).wait()
                scale(b)
                @pl.when(ci + 2 < n_chunks)
                def _():
                    start_gather(ci + 2, b)
                pltpu.async_copy(
                    outs[b], out_hbm.at[pl.ds(base + ci * chunk, chunk)],
                    wsems[b])
            return carry

        lax.fori_loop(0, n_chunks // 2, chunk_body, 0)

        for b in range(2):
            pltpu.make_async_copy(
                outs[b], out_hbm.at[pl.ds(base, chunk)], wsems[b]).wait()

    return k(idx, table)


def kernel(x, table):
    rows, cols = x.shape
    total = rows * cols
    idx = x.reshape(total).astype(jnp.int32)
    chunk = 400
    n_chunks = total // (NUM_WORKERS * chunk)
    out = _embed_sc(idx, table, chunk, n_chunks)
    return out.reshape(rows, cols, D_MODEL)
